# baseline (speedup 1.0000x reference)
import functools

import jax
import jax.numpy as jnp
from jax import lax
from jax.experimental import pallas as pl
from jax.experimental.pallas import tpu as pltpu
from jax.experimental.pallas import tpu_sc as plsc

EMBED_DIM = 64
NUM_ROWS = 16384
NUM_COLS = 200

NPIECE = 2
PIECE_ROWS = NUM_ROWS // NPIECE

NC = 2
NS = 16
NW = NC * NS
RPW = PIECE_ROWS // NW

NR = 4
NGROUPS = RPW // NR
SPLITS = ((0, 128), (128, 72))


@functools.partial(
    pl.kernel,
    mesh=plsc.VectorSubcoreMesh(core_axis_name="c", subcore_axis_name="s"),
    out_type=jax.ShapeDtypeStruct((PIECE_ROWS, NUM_COLS, EMBED_DIM),
                                  jnp.float32),
    scratch_types=[
        pltpu.VMEM((NR, NUM_COLS), jnp.int32),
        pltpu.VMEM((NR, NUM_COLS), jnp.int32),
        pltpu.VMEM((NR, NUM_COLS, EMBED_DIM), jnp.float32),
        pltpu.VMEM((NR, NUM_COLS, EMBED_DIM), jnp.float32),
        pltpu.SemaphoreType.DMA,
        pltpu.SemaphoreType.DMA,
        pltpu.SemaphoreType.DMA,
        pltpu.SemaphoreType.DMA,
        pltpu.SemaphoreType.DMA,
        pltpu.SemaphoreType.DMA,
    ],
    compiler_params=pltpu.CompilerParams(use_tc_tiling_on_sc=False),
)
def _sc_gather(data_hbm, table_hbm, out_hbm, idx0, idx1, rows0, rows1,
               isem0, isem1, gsem0, gsem1, ssem0, ssem1):
    wid = lax.axis_index("s") * NC + lax.axis_index("c")
    base = wid * RPW

    idx_b = (idx0, idx1)
    rows_b = (rows0, rows1)
    isem_b = (isem0, isem1)
    gsem_b = (gsem0, gsem1)
    ssem_b = (ssem0, ssem1)

    def idx_slice(g):
        return data_hbm.at[pl.ds(base + g * NR, NR)]

    def out_slice(g):
        return out_hbm.at[pl.ds(base + g * NR, NR)]

    pltpu.async_copy(idx_slice(0), idx0, isem0)
    pltpu.async_copy(idx_slice(1), idx1, isem1)

    def pair(gp, carry):
        for b in range(2):
            g = gp * 2 + b
            pltpu.make_async_copy(idx_slice(g), idx_b[b], isem_b[b]).wait()
            @pl.when(g >= 2)
            def _():
                pltpu.make_async_copy(rows_b[b], out_slice(g - 2),
                                      ssem_b[b]).wait()
            copies = [
                pltpu.async_copy(
                    table_hbm.at[idx_b[b].at[r, pl.ds(off, ln)]],
                    rows_b[b].at[r, pl.ds(off, ln)],
                    gsem_b[b],
                )
                for r in range(NR)
                for off, ln in SPLITS
            ]
            for cp in copies:
                cp.wait()
            @pl.when(g + 2 < NGROUPS)
            def _():
                pltpu.async_copy(idx_slice(g + 2), idx_b[b], isem_b[b])
            pltpu.async_copy(rows_b[b], out_slice(g), ssem_b[b])
        return carry

    lax.fori_loop(0, NGROUPS // 2, pair, 0)

    pltpu.make_async_copy(rows0, out_slice(NGROUPS - 2), ssem0).wait()
    pltpu.make_async_copy(rows1, out_slice(NGROUPS - 1), ssem1).wait()


def kernel(data, embedding):
    data = data.astype(jnp.int32)
    pieces = [
        _sc_gather(
            jax.lax.slice_in_dim(data, p * PIECE_ROWS, (p + 1) * PIECE_ROWS),
            embedding,
        )
        for p in range(NPIECE)
    ]
    return jnp.concatenate(pieces, axis=0)


# speedup vs baseline: 1.3051x; 1.3051x over previous
import functools

import jax
import jax.numpy as jnp
from jax import lax
from jax.experimental import pallas as pl
from jax.experimental.pallas import tpu as pltpu
from jax.experimental.pallas import tpu_sc as plsc

EMBED_DIM = 64
NUM_ROWS = 16384
NUM_COLS = 200

NC = 2
NS = 16
NW = NC * NS
RPW = NUM_ROWS // NW

NR = 4
NGROUPS = RPW // NR
SPLITS = ((0, 128), (128, 72))


@functools.partial(
    pl.kernel,
    mesh=plsc.VectorSubcoreMesh(core_axis_name="c", subcore_axis_name="s"),
    out_type=jax.ShapeDtypeStruct((NUM_ROWS, NUM_COLS, EMBED_DIM),
                                  jnp.float32),
    scratch_types=[
        pltpu.VMEM((NR, NUM_COLS), jnp.int32),
        pltpu.VMEM((NR, NUM_COLS), jnp.int32),
        pltpu.VMEM((NR, NUM_COLS, EMBED_DIM), jnp.float32),
        pltpu.VMEM((NR, NUM_COLS, EMBED_DIM), jnp.float32),
        pltpu.SemaphoreType.DMA,
        pltpu.SemaphoreType.DMA,
        pltpu.SemaphoreType.DMA,
        pltpu.SemaphoreType.DMA,
        pltpu.SemaphoreType.DMA,
        pltpu.SemaphoreType.DMA,
    ],
    compiler_params=pltpu.CompilerParams(use_tc_tiling_on_sc=False),
)
def _sc_gather(data_hbm, table_hbm, out_hbm, idx0, idx1, rows0, rows1,
               isem0, isem1, gsem0, gsem1, ssem0, ssem1):
    wid = lax.axis_index("s") * NC + lax.axis_index("c")
    base = wid * RPW

    idx_b = (idx0, idx1)
    rows_b = (rows0, rows1)
    isem_b = (isem0, isem1)
    gsem_b = (gsem0, gsem1)
    ssem_b = (ssem0, ssem1)

    def idx_slice(g):
        return data_hbm.at[pl.ds(base + g * NR, NR)]

    def out_slice(g):
        return out_hbm.at[pl.ds(base + g * NR, NR)]

    pltpu.async_copy(idx_slice(0), idx0, isem0)
    pltpu.async_copy(idx_slice(1), idx1, isem1)

    def pair(gp, carry):
        for b in range(2):
            g = gp * 2 + b
            pltpu.make_async_copy(idx_slice(g), idx_b[b], isem_b[b]).wait()
            @pl.when(g >= 2)
            def _():
                pltpu.make_async_copy(rows_b[b], out_slice(g - 2),
                                      ssem_b[b]).wait()
            copies = [
                pltpu.async_copy(
                    table_hbm.at[idx_b[b].at[r, pl.ds(off, ln)]],
                    rows_b[b].at[r, pl.ds(off, ln)],
                    gsem_b[b],
                )
                for r in range(NR)
                for off, ln in SPLITS
            ]
            for cp in copies:
                cp.wait()
            @pl.when(g + 2 < NGROUPS)
            def _():
                pltpu.async_copy(idx_slice(g + 2), idx_b[b], isem_b[b])
            pltpu.async_copy(rows_b[b], out_slice(g), ssem_b[b])
        return carry

    lax.fori_loop(0, NGROUPS // 2, pair, 0)

    pltpu.make_async_copy(rows0, out_slice(NGROUPS - 2), ssem0).wait()
    pltpu.make_async_copy(rows1, out_slice(NGROUPS - 1), ssem1).wait()


def kernel(data, embedding):
    return _sc_gather(data.astype(jnp.int32), embedding)


# speedup vs baseline: 1.3057x; 1.0005x over previous
import functools

import jax
import jax.numpy as jnp
from jax import lax
from jax.experimental import pallas as pl
from jax.experimental.pallas import tpu as pltpu
from jax.experimental.pallas import tpu_sc as plsc

EMBED_DIM = 64
NUM_ROWS = 16384
NUM_COLS = 200

NC = 2
NS = 16
NW = NC * NS
RPW = NUM_ROWS // NW

NR = 4
NGROUPS = RPW // NR
SPLITS = ((0, 128), (128, 72))


@functools.partial(
    pl.kernel,
    mesh=plsc.VectorSubcoreMesh(core_axis_name="c", subcore_axis_name="s"),
    out_type=jax.ShapeDtypeStruct((NUM_ROWS, NUM_COLS, EMBED_DIM),
                                  jnp.float32),
    scratch_types=[
        pltpu.VMEM((NR, NUM_COLS), jnp.int32),
        pltpu.VMEM((NR, NUM_COLS), jnp.int32),
        pltpu.VMEM((NR, NUM_COLS, EMBED_DIM), jnp.float32),
        pltpu.VMEM((NR, NUM_COLS, EMBED_DIM), jnp.float32),
        pltpu.SemaphoreType.DMA,
        pltpu.SemaphoreType.DMA,
        pltpu.SemaphoreType.DMA,
        pltpu.SemaphoreType.DMA,
        pltpu.SemaphoreType.DMA,
        pltpu.SemaphoreType.DMA,
    ],
    compiler_params=pltpu.CompilerParams(use_tc_tiling_on_sc=False),
)
def _sc_gather(data_hbm, table_hbm, out_hbm, idx0, idx1, rows0, rows1,
               isem0, isem1, gsem0, gsem1, ssem0, ssem1):
    wid = lax.axis_index("s") * NC + lax.axis_index("c")
    base = wid * RPW

    idx_b = (idx0, idx1)
    rows_b = (rows0, rows1)
    isem_b = (isem0, isem1)
    gsem_b = (gsem0, gsem1)
    ssem_b = (ssem0, ssem1)

    def idx_slice(g):
        return data_hbm.at[pl.ds(base + g * NR, NR)]

    def out_slice(g):
        return out_hbm.at[pl.ds(base + g * NR, NR)]

    def fire_gathers(b):
        return [
            pltpu.async_copy(
                table_hbm.at[idx_b[b].at[r, pl.ds(off, ln)]],
                rows_b[b].at[r, pl.ds(off, ln)],
                gsem_b[b],
            )
            for r in range(NR)
            for off, ln in SPLITS
        ]

    def wait_gathers(b):
        for r in range(NR):
            for off, ln in SPLITS:
                pltpu.make_async_copy(
                    table_hbm.at[idx_b[b].at[r, pl.ds(off, ln)]],
                    rows_b[b].at[r, pl.ds(off, ln)],
                    gsem_b[b],
                ).wait()

    pltpu.async_copy(idx_slice(0), idx0, isem0)

    def pair(gp, carry):
        for b in range(2):
            g = gp * 2 + b
            o = 1 - b
            @pl.when(g >= 2)
            def _():
                pltpu.make_async_copy(rows_b[b], out_slice(g - 2),
                                      ssem_b[b]).wait()
            pltpu.make_async_copy(idx_slice(g), idx_b[b], isem_b[b]).wait()
            fire_gathers(b)
            @pl.when(g >= 1)
            def _():
                wait_gathers(o)
                pltpu.async_copy(rows_b[o], out_slice(g - 1), ssem_b[o])
            @pl.when(g + 1 < NGROUPS)
            def _():
                pltpu.async_copy(idx_slice(g + 1), idx_b[o], isem_b[o])
        return carry

    lax.fori_loop(0, NGROUPS // 2, pair, 0)

    wait_gathers(1)
    pltpu.async_copy(rows1, out_slice(NGROUPS - 1), ssem1)
    pltpu.make_async_copy(rows0, out_slice(NGROUPS - 2), ssem0).wait()
    pltpu.make_async_copy(rows1, out_slice(NGROUPS - 1), ssem1).wait()


def kernel(data, embedding):
    return _sc_gather(data.astype(jnp.int32), embedding)
